# baseline (speedup 1.0000x reference)
import jax
import jax.numpy as jnp
from jax import lax
from jax.experimental import pallas as pl
from jax.experimental.pallas import tpu as pltpu


def _make_divisible(v, d=8):
    nv = max(d, int(v + d / 2) // d * d)
    if nv < 0.9 * v:
        nv += d
    return nv

_BASE, _MULT = 32, 1.8
_CPS = [_make_divisible(_BASE)] + [_make_divisible(_BASE * _MULT ** s) for s in range(3)]
_STRIDES = {2: (2, 2), 4: (2, 1)}
_BLOCKS = []
_bid, _cin = 1, _CPS[0]
for _cout, _n in [(_CPS[1], 3), (_CPS[2], 2), (_CPS[3], 1)]:
    for _ in range(_n):
        _BLOCKS.append((_bid, _cin, _cout, _STRIDES.get(_bid, (1, 1))))
        _cin = _cout
        _bid += 1

_EPS = 1e-5


def _pad_hw(x, extra_row=0):
    H, W, C = x.shape
    zr = jnp.zeros((1, W, C), jnp.float32)
    x = jnp.concatenate([zr, x, zr] + [zr] * extra_row, axis=0)
    zc = jnp.zeros((x.shape[0], 1, C), jnp.float32)
    return jnp.concatenate([zc, x, zc], axis=1)


def _taps(x3, stride):
    H, W, C = x3.shape
    sh, sw = stride
    Ho = (H - 1) // sh + 1 if sh == 2 else H
    Wo = (W - 1) // sw + 1 if sw == 2 else W
    if sh == 1:
        xp = _pad_hw(x3)
        return Ho, Wo, [xp[i:i + Ho, j:j + Wo, :] for i in range(3) for j in range(3)]
    Ho = (H + 2 - 3) // 2 + 1
    Wo = (W + 2 - 3) // 2 + 1 if sw == 2 else W
    xp = _pad_hw(x3, extra_row=(H + 2) % 2)
    x4 = xp.reshape(xp.shape[0] // 2, 2, xp.shape[1], C)
    taps = []
    for i in range(3):
        rows = x4[i // 2:i // 2 + Ho, i % 2]
        if sw == 1:
            taps.extend(rows[:, j:j + Wo, :] for j in range(3))
        else:
            r4 = rows.reshape(Ho, xp.shape[1] // 2, 2, C)
            ev = r4[:, :, 0, :]
            od = r4[:, :, 1, :]
            taps.extend([ev[:, 0:Wo, :], od[:, 0:Wo, :], ev[:, 1:Wo + 1, :]])
    return Ho, Wo, taps


def _norm_affine(y, n_valid, g_row, b_row, maskf):
    s = jnp.sum(y, axis=0, keepdims=True)
    sq = jnp.sum(y * y, axis=0, keepdims=True)
    mean = s / n_valid
    var = (sq - n_valid * mean * mean) / (n_valid - 1.0)
    scale = g_row * lax.rsqrt(var + _EPS)
    shift = b_row - mean * scale
    out = y * scale + shift
    if maskf is not None:
        out = out * maskf
    return out


def _network_kernel(a4_ref, dev_ref, w04_ref, w1p_ref, wb_ref, gb_ref, o_ref):
    f32 = jnp.float32
    G = 4
    dvals = [dev_ref[g, 0, 0] for g in range(G)]

    goff = [0]

    def grow(C):
        o = goff[0]
        goff[0] += 128
        g6 = gb_ref[0, :, o:o + 128]
        b6 = gb_ref[1, :, o:o + 128]
        outs_g, outs_b = [], []
        for g in range(G):
            sel = (lax.broadcasted_iota(jnp.int32, (6, 128), 0) == dvals[g]).astype(f32)
            outs_g.append(jnp.sum(g6 * sel, axis=0, keepdims=True))
            outs_b.append(jnp.sum(b6 * sel, axis=0, keepdims=True))
        return jnp.concatenate(outs_g, axis=1), jnp.concatenate(outs_b, axis=1)

    woff = [0]

    def wmat(K, N):
        o = woff[0]
        woff[0] += 128
        return wb_ref[o:o + 128, 0:128]

    def wdw(C):
        o = woff[0]
        woff[0] += 9
        return wb_ref[o:o + 9, :]

    def gdot(a, w):
        return jnp.concatenate(
            [jnp.dot(a[:, 128 * g:128 * (g + 1)], w, preferred_element_type=f32)
             for g in range(G)], axis=1)

    z0 = jnp.dot(a4_ref[0], w04_ref[...], preferred_element_type=f32)
    h14 = jnp.maximum(z0, 0.0).reshape(4, 64, 40, 32)
    cols = []
    for i in range(3):
        for j in range(3):
            s = h14[(i % 2) * 2 + (j % 2), i // 2:i // 2 + 63, j // 2:j // 2 + 32, :]
            cols.append(s.reshape(63 * 32, 32))
    A1 = jnp.concatenate(cols, axis=1)
    z1 = jnp.dot(A1, w1p_ref[...], preferred_element_type=f32)
    rowm = (lax.broadcasted_iota(jnp.int32, (2016, 1), 0) % 32 < 31).astype(f32)
    h = jnp.maximum(z1, 0.0) * rowm

    H, W, n, mk = 63, 32, 1953.0, rowm
    for bid, cin, cout, stride in _BLOCKS:
        exp_c = 64 if cin == 32 else 120

        ge, be = grow(exp_c)
        y = gdot(h, wmat(cin, exp_c))
        y = _norm_affine(y, n, ge, be, mk)
        y = jnp.maximum(y, 0.0)
        if mk is not None:
            y = y * mk

        wd0 = wdw(exp_c)
        gd, bd = grow(exp_c)
        Ho, Wo, taps = _taps(y.reshape(H, W, G * 128), stride)
        acc = taps[0] * wd0[0, :][None, None, :]
        for t in range(1, 9):
            acc = acc + taps[t] * wd0[t, :][None, None, :]
        Ro = Ho * Wo
        if stride == (1, 1):
            no, mko = n, mk
        else:
            no, mko = float(Ro), None
        yd = acc.reshape(Ro, G * 128)
        if mko is not None:
            yd = yd * mko
        yd = _norm_affine(yd, no, gd, bd, mko)
        yd = jnp.maximum(yd, 0.0)

        gp, bp = grow(cout)
        z = gdot(yd, wmat(exp_c, cout))
        z = _norm_affine(z, no, gp, bp, mko)

        if cin == cout and stride == (1, 1):
            sc = h
        else:
            if stride != (1, 1):
                _, _, ptaps = _taps(h.reshape(H, W, G * 128), stride)
                pacc = ptaps[0]
                for t in range(1, 9):
                    pacc = pacc + ptaps[t]
                sc = (pacc / 9.0).reshape(Ro, G * 128)
            else:
                sc = h
            if cin != cout:
                gs, bs = grow(cout)
                sc = gdot(sc, wmat(cin, cout))
                sc = _norm_affine(sc, no, gs, bs, mko)

        h = jnp.maximum(z + sc, 0.0)
        if mko is not None:
            h = h * mko
        H, W, n, mk = Ho, Wo, no, mko

    gf, bf = grow(10)
    zf = gdot(h, wmat(104, 10))
    zn = _norm_affine(zf, 256.0, gf, bf, None)
    o_ref[0] = jnp.sum(zn, axis=0, keepdims=True) / 256.0


def kernel(x, device_ids, params):
    B = x.shape[0]
    G = 4
    f32 = jnp.float32

    xp = jnp.pad(x[:, 0], ((0, 0), (0, 8), (0, 40)))
    planes = []
    for p in range(2):
        for q in range(2):
            for i in range(3):
                for j in range(3):
                    planes.append(
                        lax.slice(xp, (0, 2 * p + i, 2 * q + j), (B, 2 * p + i + 253, 2 * q + j + 157),
                                  (1, 4, 4)))
    a4 = jnp.stack(planes, axis=0).reshape(2, 2, 3, 3, B, 64, 40)
    a4 = a4.transpose(4, 0, 1, 5, 6, 2, 3).reshape(B, 10240, 9)
    a4 = a4.reshape(B // G, G, 10240, 9).transpose(0, 2, 1, 3).reshape(B // G, 10240, 36)

    w0t = params['in0_w'][:, 0].transpose(1, 2, 0).reshape(9, 8)
    w04 = jnp.zeros((36, 32), f32)
    for g in range(G):
        w04 = w04.at[g * 9:(g + 1) * 9, g * 8:(g + 1) * 8].set(w0t)
    w1k = params['in1_w'].transpose(2, 3, 1, 0).reshape(72, 32)
    w1p = jnp.zeros((288, 512), f32)
    for t in range(9):
        for g in range(G):
            w1p = w1p.at[t * 32 + g * 8:t * 32 + (g + 1) * 8, g * 128:g * 128 + 32].set(
                w1k[t * 8:(t + 1) * 8, :])

    slabs, gs, bs = [], [], []

    def put(a, diag=True):
        K, N = a.shape
        if diag:
            a = jnp.pad(a, ((0, 128 - K), (0, 128 - N)))
        else:
            a = jnp.tile(jnp.pad(a, ((0, 0), (0, 128 - N))), (1, G))
        slabs.append(a)

    def putg(g, b):
        gs.append(jnp.pad(g[:, :, 0, 0], ((0, 0), (0, 128 - g.shape[1]))))
        bs.append(jnp.pad(b[:, :, 0, 0], ((0, 0), (0, 128 - b.shape[1]))))

    for bid, cin, cout, stride in _BLOCKS:
        nm = f'b{bid}'
        put(params[nm + '_exp_w'][:, :, 0, 0].T)
        putg(params[nm + '_exp_g'], params[nm + '_exp_b'])
        put(params[nm + '_dw_w'][:, 0].transpose(1, 2, 0).reshape(9, -1), diag=False)
        putg(params[nm + '_dw_g'], params[nm + '_dw_b'])
        put(params[nm + '_proj_w'][:, :, 0, 0].T)
        putg(params[nm + '_proj_g'], params[nm + '_proj_b'])
        if cin != cout:
            put(params[nm + '_sc_w'][:, :, 0, 0].T)
            putg(params[nm + '_sc_g'], params[nm + '_sc_b'])
    put(params['ff_w'][:, :, 0, 0].T)
    putg(params['ff_g'], params['ff_b'])

    wb = jnp.concatenate([jnp.pad(s, ((0, 0), (0, 512 - s.shape[1]))) for s in slabs], axis=0)
    gall = jnp.concatenate(gs, axis=1)
    ball = jnp.concatenate(bs, axis=1)
    gb = jnp.stack([gall, ball], axis=0)

    dev3 = device_ids.reshape(B, 1, 1)

    out = pl.pallas_call(
        _network_kernel,
        grid=(B // G,),
        in_specs=[
            pl.BlockSpec((1, 10240, 36), lambda i: (i, 0, 0)),
            pl.BlockSpec((G, 1, 1), lambda i: (i, 0, 0)),
            pl.BlockSpec(w04.shape, lambda i: (0, 0)),
            pl.BlockSpec(w1p.shape, lambda i: (0, 0)),
            pl.BlockSpec(wb.shape, lambda i: (0, 0)),
            pl.BlockSpec(gb.shape, lambda i: (0, 0, 0)),
        ],
        out_specs=pl.BlockSpec((1, 1, G * 128), lambda i: (i, 0, 0)),
        out_shape=jax.ShapeDtypeStruct((B // G, 1, G * 128), f32),
        compiler_params=pltpu.CompilerParams(
            dimension_semantics=("parallel",),
            vmem_limit_bytes=56 * 1024 * 1024,
        ),
        name="fused_dcase_net_g4",
    )(a4, dev3, w04, w1p, wb, gb)
    return out.reshape(B, 128)[:, :10]


# speedup vs baseline: 1.2593x; 1.2593x over previous
import jax
import jax.numpy as jnp
from jax import lax
from jax.experimental import pallas as pl
from jax.experimental.pallas import tpu as pltpu


def _make_divisible(v, d=8):
    nv = max(d, int(v + d / 2) // d * d)
    if nv < 0.9 * v:
        nv += d
    return nv

_BASE, _MULT = 32, 1.8
_CPS = [_make_divisible(_BASE)] + [_make_divisible(_BASE * _MULT ** s) for s in range(3)]
_STRIDES = {2: (2, 2), 4: (2, 1)}
_BLOCKS = []
_bid, _cin = 1, _CPS[0]
for _cout, _n in [(_CPS[1], 3), (_CPS[2], 2), (_CPS[3], 1)]:
    for _ in range(_n):
        _BLOCKS.append((_bid, _cin, _cout, _STRIDES.get(_bid, (1, 1))))
        _cin = _cout
        _bid += 1

_EPS = 1e-5


def _pad_hw(x, extra_row=0):
    H, W, C = x.shape
    zr = jnp.zeros((1, W, C), x.dtype)
    x = jnp.concatenate([zr, x, zr] + [zr] * extra_row, axis=0)
    zc = jnp.zeros((x.shape[0], 1, C), x.dtype)
    return jnp.concatenate([zc, x, zc], axis=1)


def _taps(x3, stride):
    H, W, C = x3.shape
    sh, sw = stride
    Ho = (H - 1) // sh + 1 if sh == 2 else H
    Wo = (W - 1) // sw + 1 if sw == 2 else W
    if sh == 1:
        xp = _pad_hw(x3)
        return Ho, Wo, [xp[i:i + Ho, j:j + Wo, :] for i in range(3) for j in range(3)]
    Ho = (H + 2 - 3) // 2 + 1
    Wo = (W + 2 - 3) // 2 + 1 if sw == 2 else W
    xp = _pad_hw(x3, extra_row=(H + 2) % 2)
    x4 = xp.reshape(xp.shape[0] // 2, 2, xp.shape[1], C)
    taps = []
    for i in range(3):
        rows = x4[i // 2:i // 2 + Ho, i % 2]
        if sw == 1:
            taps.extend(rows[:, j:j + Wo, :] for j in range(3))
        else:
            r4 = rows.reshape(Ho, xp.shape[1] // 2, 2, C)
            ev = r4[:, :, 0, :]
            od = r4[:, :, 1, :]
            taps.extend([ev[:, 0:Wo, :], od[:, 0:Wo, :], ev[:, 1:Wo + 1, :]])
    return Ho, Wo, taps


def _norm_affine(y, n_valid, g_row, b_row, maskf):
    s = jnp.sum(y, axis=0, keepdims=True)
    sq = jnp.sum(y * y, axis=0, keepdims=True)
    mean = s / n_valid
    var = (sq - n_valid * mean * mean) / (n_valid - 1.0)
    scale = g_row * lax.rsqrt(var + _EPS)
    shift = b_row - mean * scale
    out = y * scale + shift
    if maskf is not None:
        out = out * maskf
    return out


def _network_kernel(a4_ref, dev_ref, w04_ref, w1p_ref, wb_ref, gb_ref, o_ref):
    f32 = jnp.float32
    G = 4
    dvals = [dev_ref[g, 0, 0] for g in range(G)]

    goff = [0]

    def grow(C):
        o = goff[0]
        goff[0] += C
        g6 = gb_ref[0, :, o:o + C]
        b6 = gb_ref[1, :, o:o + C]
        outs_g, outs_b = [], []
        for g in range(G):
            sel = (lax.broadcasted_iota(jnp.int32, (6, C), 0) == dvals[g]).astype(f32)
            outs_g.append(jnp.sum(g6 * sel, axis=0, keepdims=True))
            outs_b.append(jnp.sum(b6 * sel, axis=0, keepdims=True))
        return jnp.concatenate(outs_g, axis=1), jnp.concatenate(outs_b, axis=1)

    woff = [0]

    def wmat(K, N):
        o = woff[0]
        woff[0] += G * K
        return wb_ref[o:o + G * K, 0:G * N]

    def wdw(C):
        o = woff[0]
        woff[0] += 9
        return wb_ref[o:o + 9, 0:G * C]

    z0 = jnp.dot(a4_ref[0], w04_ref[...], preferred_element_type=f32)
    h14 = jnp.maximum(z0, 0.0).reshape(4, 64, 40, 32)
    cols = []
    for i in range(3):
        for j in range(3):
            s = h14[(i % 2) * 2 + (j % 2), i // 2:i // 2 + 63, j // 2:j // 2 + 32, :]
            cols.append(s.reshape(63 * 32, 32))
    A1 = jnp.concatenate(cols, axis=1)
    z1 = jnp.dot(A1, w1p_ref[...], preferred_element_type=f32)
    rowm = (lax.broadcasted_iota(jnp.int32, (2016, 1), 0) % 32 < 31).astype(f32)
    h = jnp.maximum(z1, 0.0) * rowm

    H, W, n, mk = 63, 32, 1953.0, rowm
    for bid, cin, cout, stride in _BLOCKS:
        exp_c = 64 if cin == 32 else 120

        ge, be = grow(exp_c)
        y = jnp.dot(h, wmat(cin, exp_c), preferred_element_type=f32)
        y = _norm_affine(y, n, ge, be, mk)
        y = jnp.maximum(y, 0.0)
        if mk is not None:
            y = y * mk

        wd0 = wdw(exp_c)
        gd, bd = grow(exp_c)
        Ho, Wo, taps = _taps(y.astype(jnp.bfloat16).reshape(H, W, G * exp_c), stride)
        wdb = wd0.astype(jnp.bfloat16)
        acc = taps[0] * wdb[0, :][None, None, :]
        for t in range(1, 9):
            acc = acc + taps[t] * wdb[t, :][None, None, :]
        acc = acc.astype(jnp.float32)
        Ro = Ho * Wo
        if stride == (1, 1):
            no, mko = n, mk
        else:
            no, mko = float(Ro), None
        yd = acc.reshape(Ro, G * exp_c)
        if mko is not None:
            yd = yd * mko
        yd = _norm_affine(yd, no, gd, bd, mko)
        yd = jnp.maximum(yd, 0.0)

        gp, bp = grow(cout)
        z = jnp.dot(yd, wmat(exp_c, cout), preferred_element_type=f32)
        z = _norm_affine(z, no, gp, bp, mko)

        if cin == cout and stride == (1, 1):
            sc = h
        else:
            if stride != (1, 1):
                _, _, ptaps = _taps(h.astype(jnp.bfloat16).reshape(H, W, G * cin), stride)
                pacc = ptaps[0]
                for t in range(1, 9):
                    pacc = pacc + ptaps[t]
                sc = (pacc.astype(jnp.float32) / 9.0).reshape(Ro, G * cin)
            else:
                sc = h
            if cin != cout:
                gs, bs = grow(cout)
                sc = jnp.dot(sc, wmat(cin, cout), preferred_element_type=f32)
                sc = _norm_affine(sc, no, gs, bs, mko)

        h = jnp.maximum(z + sc, 0.0)
        if mko is not None:
            h = h * mko
        H, W, n, mk = Ho, Wo, no, mko

    gf, bf = grow(10)
    zf = jnp.dot(h, wmat(104, 10), preferred_element_type=f32)
    zn = _norm_affine(zf, 256.0, gf, bf, None)
    o_ref[0] = jnp.sum(zn, axis=0, keepdims=True) / 256.0


def kernel(x, device_ids, params):
    B = x.shape[0]
    G = 4
    f32 = jnp.float32

    xp = jnp.pad(x[:, 0], ((0, 0), (0, 8), (0, 40)))
    planes = []
    for p in range(2):
        for q in range(2):
            for i in range(3):
                for j in range(3):
                    planes.append(
                        lax.slice(xp, (0, 2 * p + i, 2 * q + j), (B, 2 * p + i + 253, 2 * q + j + 157),
                                  (1, 4, 4)))
    a4 = jnp.stack(planes, axis=0).reshape(2, 2, 3, 3, B, 64, 40)
    a4 = a4.transpose(4, 0, 1, 5, 6, 2, 3).reshape(B, 10240, 9)
    a4 = a4.reshape(B // G, G, 10240, 9).transpose(0, 2, 1, 3).reshape(B // G, 10240, 36)

    w0t = params['in0_w'][:, 0].transpose(1, 2, 0).reshape(9, 8)
    w04 = jnp.zeros((36, 32), f32)
    for g in range(G):
        w04 = w04.at[g * 9:(g + 1) * 9, g * 8:(g + 1) * 8].set(w0t)
    w1k = params['in1_w'].transpose(2, 3, 1, 0).reshape(72, 32)
    w1p = jnp.zeros((288, 128), f32)
    for t in range(9):
        for g in range(G):
            w1p = w1p.at[t * 32 + g * 8:t * 32 + (g + 1) * 8, g * 32:(g + 1) * 32].set(
                w1k[t * 8:(t + 1) * 8, :])

    slabs, gs, bs = [], [], []

    def bd(a):
        K, N = a.shape
        z = jnp.zeros((G * K, G * N), f32)
        for g in range(G):
            z = z.at[g * K:(g + 1) * K, g * N:(g + 1) * N].set(a)
        return z

    def put(a, diag=True):
        a = bd(a) if diag else jnp.tile(a, (1, G))
        slabs.append(jnp.pad(a, ((0, 0), (0, (-a.shape[1]) % 128))))

    def putg(g, b):
        gs.append(g[:, :, 0, 0])
        bs.append(b[:, :, 0, 0])

    for bid, cin, cout, stride in _BLOCKS:
        nm = f'b{bid}'
        put(params[nm + '_exp_w'][:, :, 0, 0].T)
        putg(params[nm + '_exp_g'], params[nm + '_exp_b'])
        put(params[nm + '_dw_w'][:, 0].transpose(1, 2, 0).reshape(9, -1), diag=False)
        putg(params[nm + '_dw_g'], params[nm + '_dw_b'])
        put(params[nm + '_proj_w'][:, :, 0, 0].T)
        putg(params[nm + '_proj_g'], params[nm + '_proj_b'])
        if cin != cout:
            put(params[nm + '_sc_w'][:, :, 0, 0].T)
            putg(params[nm + '_sc_g'], params[nm + '_sc_b'])
    put(params['ff_w'][:, :, 0, 0].T)
    putg(params['ff_g'], params['ff_b'])

    KT = max(s.shape[1] for s in slabs)
    wb = jnp.concatenate([jnp.pad(s, ((0, 0), (0, KT - s.shape[1]))) for s in slabs], axis=0)
    gall = jnp.concatenate(gs, axis=1)
    ball = jnp.concatenate(bs, axis=1)
    gb = jnp.stack([gall, ball], axis=0)

    dev3 = device_ids.reshape(B, 1, 1)

    out = pl.pallas_call(
        _network_kernel,
        grid=(B // G,),
        in_specs=[
            pl.BlockSpec((1, 10240, 36), lambda i: (i, 0, 0)),
            pl.BlockSpec((G, 1, 1), lambda i: (i, 0, 0)),
            pl.BlockSpec(w04.shape, lambda i: (0, 0)),
            pl.BlockSpec(w1p.shape, lambda i: (0, 0)),
            pl.BlockSpec(wb.shape, lambda i: (0, 0)),
            pl.BlockSpec(gb.shape, lambda i: (0, 0, 0)),
        ],
        out_specs=pl.BlockSpec((1, 1, G * 10), lambda i: (i, 0, 0)),
        out_shape=jax.ShapeDtypeStruct((B // G, 1, G * 10), f32),
        compiler_params=pltpu.CompilerParams(
            dimension_semantics=("parallel",),
            vmem_limit_bytes=56 * 1024 * 1024,
        ),
        name="fused_dcase_net_g4",
    )(a4, dev3, w04, w1p, wb, gb)
    return out.reshape(B, 10)
